# baseline (speedup 1.0000x reference)
import functools
import jax
import jax.numpy as jnp
from jax.experimental import pallas as pl
from jax.experimental.pallas import tpu as pltpu

_N = 100000
_B = 2
_R = 2000


def _row_spec(c):
    return pl.BlockSpec((_R, c), lambda i: (i, 0))


def _full_spec(shape):
    nd = len(shape)
    return pl.BlockSpec(shape, lambda i: (0,) * nd)


def _stage0_body(x_ref, deg_ref, encW_ref, encb_ref, gcnW_ref, resW_ref,
                 resb_ref, g_ref, id_ref):
    dinv = jax.lax.rsqrt(deg_ref[...] + 1.0)
    h = jnp.maximum(
        jax.lax.dot_general(x_ref[...], encW_ref[...], (((1,), (0,)), ((), ())),
                            preferred_element_type=jnp.float32) + encb_ref[...],
        0.0)
    g_ref[...] = dinv * jax.lax.dot_general(
        h, gcnW_ref[...], (((1,), (0,)), ((), ())),
        preferred_element_type=jnp.float32)
    id_ref[...] = jax.lax.dot_general(
        h, resW_ref[...], (((1,), (0,)), ((), ())),
        preferred_element_type=jnp.float32) + resb_ref[...]


def _stage_mid_body(scat_ref, g_ref, idc_ref, deg_ref, bprev_ref, gcnW_ref,
                    resW_ref, resb_ref, gout_ref, idout_ref):
    dinv = jax.lax.rsqrt(deg_ref[...] + 1.0)
    h = jnp.maximum(
        dinv * (scat_ref[...] + g_ref[...]) + bprev_ref[...] + idc_ref[...],
        0.0)
    gout_ref[...] = dinv * jax.lax.dot_general(
        h, gcnW_ref[...], (((1,), (0,)), ((), ())),
        preferred_element_type=jnp.float32)
    idout_ref[...] = jax.lax.dot_general(
        h, resW_ref[...], (((1,), (0,)), ((), ())),
        preferred_element_type=jnp.float32) + resb_ref[...]


def _stage_final_body(scat_ref, g_ref, idc_ref, deg_ref, bprev_ref, d1W_ref,
                      d1b_ref, d2W_ref, d2b_ref, out_ref):
    dinv = jax.lax.rsqrt(deg_ref[...] + 1.0)
    h = jnp.maximum(
        dinv * (scat_ref[...] + g_ref[...]) + bprev_ref[...] + idc_ref[...],
        0.0)
    p = jnp.maximum(
        jax.lax.dot_general(h, d1W_ref[...], (((1,), (0,)), ((), ())),
                            preferred_element_type=jnp.float32) + d1b_ref[...],
        0.0)
    out_ref[...] = jax.lax.dot_general(
        p, d2W_ref[...], (((1,), (0,)), ((), ())),
        preferred_element_type=jnp.float32) + d2b_ref[...]


def _stage0(x2, deg2, enc_W, enc_b, gcn_W0, res_W0, res_b0):
    n2 = x2.shape[0]
    c_in, c_h = enc_W.shape
    c_out = gcn_W0.shape[1]
    return pl.pallas_call(
        _stage0_body,
        grid=(n2 // _R,),
        in_specs=[
            _row_spec(c_in), _row_spec(1),
            _full_spec(enc_W.shape), _full_spec((1, c_h)),
            _full_spec(gcn_W0.shape), _full_spec(res_W0.shape),
            _full_spec((1, c_out)),
        ],
        out_specs=[_row_spec(c_out), _row_spec(c_out)],
        out_shape=[
            jax.ShapeDtypeStruct((n2, c_out), jnp.float32),
            jax.ShapeDtypeStruct((n2, c_out), jnp.float32),
        ],
    )(x2, deg2, enc_W, enc_b.reshape(1, -1), gcn_W0, res_W0,
      res_b0.reshape(1, -1))


def _stage_mid(scat, g, idc, deg2, b_prev, gcn_Wi, res_Wi, res_bi):
    n2, c_in = g.shape
    c_out = gcn_Wi.shape[1]
    return pl.pallas_call(
        _stage_mid_body,
        grid=(n2 // _R,),
        in_specs=[
            _row_spec(c_in), _row_spec(c_in), _row_spec(c_in), _row_spec(1),
            _full_spec((1, c_in)),
            _full_spec(gcn_Wi.shape), _full_spec(res_Wi.shape),
            _full_spec((1, c_out)),
        ],
        out_specs=[_row_spec(c_out), _row_spec(c_out)],
        out_shape=[
            jax.ShapeDtypeStruct((n2, c_out), jnp.float32),
            jax.ShapeDtypeStruct((n2, c_out), jnp.float32),
        ],
    )(scat, g, idc, deg2, b_prev.reshape(1, -1), gcn_Wi, res_Wi,
      res_bi.reshape(1, -1))


def _stage_final(scat, g, idc, deg2, b_prev, d1_W, d1_b, d2_W, d2_b):
    n2, c_in = g.shape
    return pl.pallas_call(
        _stage_final_body,
        grid=(n2 // _R,),
        in_specs=[
            _row_spec(c_in), _row_spec(c_in), _row_spec(c_in), _row_spec(1),
            _full_spec((1, c_in)),
            _full_spec(d1_W.shape), _full_spec((1, d1_W.shape[1])),
            _full_spec(d2_W.shape), _full_spec((1, 1)),
        ],
        out_specs=[_row_spec(1)],
        out_shape=[jax.ShapeDtypeStruct((n2, 1), jnp.float32)],
    )(scat, g, idc, deg2, b_prev.reshape(1, -1), d1_W, d1_b.reshape(1, -1),
      d2_W, d2_b.reshape(1, -1))[0]


def kernel(x, edge_index, edge_weight, enc_W, enc_b, gcn_W, gcn_b, res_W,
           res_b, d1_W, d1_b, d2_W, d2_b):
    src, dst = edge_index[0], edge_index[1]
    src2 = jnp.concatenate([src, src + _N])
    dst2 = jnp.concatenate([dst, dst + _N])
    ew2 = jnp.concatenate([edge_weight, edge_weight])

    deg = jnp.zeros((_N,), jnp.float32).at[dst].add(edge_weight)
    deg2 = jnp.tile(deg, 2).reshape(2 * _N, 1)

    x2 = x.reshape(2 * _N, x.shape[-1])
    g, idc = _stage0(x2, deg2, enc_W, enc_b, gcn_W[0], res_W[0], res_b[0])

    for i in range(3):
        msg = ew2[:, None] * g[src2]
        scat = jnp.zeros(g.shape, jnp.float32).at[dst2].add(msg)
        if i < 2:
            g, idc = _stage_mid(scat, g, idc, deg2, gcn_b[i], gcn_W[i + 1],
                                res_W[i + 1], res_b[i + 1])
        else:
            pred = _stage_final(scat, g, idc, deg2, gcn_b[i], d1_W, d1_b,
                                d2_W, d2_b)
    return pred.reshape(_B, _N)


# speedup vs baseline: 16.2757x; 16.2757x over previous
import functools
import jax
import jax.numpy as jnp
from jax import lax
from jax.experimental import pallas as pl
from jax.experimental.pallas import tpu as pltpu
from jax.experimental.pallas import tpu_sc as plsc

_N = 100000
_B = 2
_R = 2000

_E = 1600000
_NC, _NS = 2, 16
_HALF = _N // _NC
_TROWS = 3128
_ACCR = _NS * _TROWS
_ZR = 184
_TROWS_D = 3128
_ACCR_D = _NS * _TROWS_D
_BLK = 2048
_SUB = 128
_EPT = 102400
_EPAD = _NS * _EPT


def _sc_mesh():
    return plsc.VectorSubcoreMesh(core_axis_name="c", subcore_axis_name="s",
                                  num_cores=_NC, num_subcores=_NS)


def _scatter_rows_body(c, g_h, src_h, dst_h, ew_h, z_h, out_h, src_v, dst_v,
                       ew_v, rows_v, dstl_v, zb_v, acc, sem):
    cid = lax.axis_index("c")
    sid = lax.axis_index("s")
    base = cid * _HALF
    tbase = sid * _EPT
    iota = lax.iota(jnp.int32, 16)

    pltpu.sync_copy(z_h, zb_v)
    for k in range(_TROWS // _ZR):
        pltpu.sync_copy(zb_v, acc.at[pl.ds(sid * _TROWS + k * _ZR, _ZR)])
    plsc.subcore_barrier()

    def blk_body(j, _):
        off = tbase + j * _BLK
        pltpu.sync_copy(src_h.at[pl.ds(off, _BLK)], src_v)
        pltpu.sync_copy(dst_h.at[pl.ds(off, _BLK)], dst_v)
        pltpu.sync_copy(ew_h.at[pl.ds(off, _BLK)], ew_v.at[pl.ds(0, _BLK)])

        def sub_body(s, _):
            so = s * _SUB
            pltpu.async_copy(g_h.at[src_v.at[pl.ds(so, _SUB)]], rows_v,
                             sem).wait()
            for j16 in range(_SUB // 16):
                d16 = dst_v[pl.ds(so + j16 * 16, 16)] - base
                oob = (d16 < 0) | (d16 >= _HALF)
                dstl_v[pl.ds(j16 * 16, 16)] = jnp.where(oob, _HALF + iota, d16)
            def mul_body(r, _):
                w = ew_v[pl.ds(so + r, 16)][0]
                for c16 in range(c // 16):
                    sl = pl.ds(c16 * 16, 16)
                    rows_v[r, sl] = rows_v[r, sl] * w
                return ()

            lax.fori_loop(0, _SUB, mul_body, (), unroll=8)
            pltpu.sync_copy(rows_v, acc.at[dstl_v], add=True)
            return ()

        lax.fori_loop(0, _BLK // _SUB, sub_body, (), unroll=False)
        return ()

    lax.fori_loop(0, _EPT // _BLK, blk_body, (), unroll=False)
    plsc.subcore_barrier()

    rb = sid * _TROWS
    last = _NS - 1
    nchunk = _TROWS // _ZR
    last_full = (_HALF - last * _TROWS) // _ZR
    tail_rows = _HALF - last * _TROWS - last_full * _ZR
    for k in range(nchunk):
        off = rb + k * _ZR
        if k < last_full:
            pltpu.sync_copy(acc.at[pl.ds(off, _ZR)], zb_v)
            pltpu.sync_copy(zb_v, out_h.at[pl.ds(base + off, _ZR)])
        elif k == last_full:

            @pl.when(sid < last)
            def _():
                pltpu.sync_copy(acc.at[pl.ds(off, _ZR)], zb_v)
                pltpu.sync_copy(zb_v, out_h.at[pl.ds(base + off, _ZR)])

            @pl.when(sid == last)
            def _():
                pltpu.sync_copy(acc.at[pl.ds(off, tail_rows)],
                                zb_v.at[pl.ds(0, tail_rows)])
                pltpu.sync_copy(zb_v.at[pl.ds(0, tail_rows)],
                                out_h.at[pl.ds(base + off, tail_rows)])
        else:

            @pl.when(sid < last)
            def _():
                pltpu.sync_copy(acc.at[pl.ds(off, _ZR)], zb_v)
                pltpu.sync_copy(zb_v, out_h.at[pl.ds(base + off, _ZR)])


def _sc_scatter_rows(g, srcp, dstp, ewp, c):
    zrows = jnp.zeros((_ZR, c), jnp.float32)
    kfn = functools.partial(
        pl.kernel,
        out_type=jax.ShapeDtypeStruct((_N, c), jnp.float32),
        mesh=_sc_mesh(),
        scratch_types=[
            pltpu.VMEM((_BLK,), jnp.int32),
            pltpu.VMEM((_BLK,), jnp.int32),
            pltpu.VMEM((_BLK + 16,), jnp.float32),
            pltpu.VMEM((_SUB, c), jnp.float32),
            pltpu.VMEM((_SUB,), jnp.int32),
            pltpu.VMEM((_ZR, c), jnp.float32),
            pltpu.VMEM_SHARED((_ACCR, c), jnp.float32),
            pltpu.SemaphoreType.DMA,
        ],
        compiler_params=pltpu.CompilerParams(use_tc_tiling_on_sc=False),
    )(functools.partial(_scatter_rows_body, c))
    return kfn(g, srcp, dstp, ewp, zrows)


def _deg_body(dst_h, ew_h, z_h, out_h, dst_v, ew_v, dstl_v, zb_v, acc, sem):
    del sem
    cid = lax.axis_index("c")
    sid = lax.axis_index("s")
    base = cid * _HALF
    tbase = sid * _EPT
    iota = lax.iota(jnp.int32, 16)

    pltpu.sync_copy(z_h, zb_v)
    pltpu.sync_copy(zb_v, acc.at[pl.ds(sid * _TROWS_D, _TROWS_D)])
    plsc.subcore_barrier()

    def blk_body(j, _):
        off = tbase + j * _BLK
        pltpu.sync_copy(dst_h.at[pl.ds(off, _BLK)], dst_v)
        pltpu.sync_copy(ew_h.at[pl.ds(off, _BLK)], ew_v)

        def sub_body(s, _):
            so = s * _SUB
            for j16 in range(_SUB // 16):
                d16 = dst_v[pl.ds(so + j16 * 16, 16)] - base
                oob = (d16 < 0) | (d16 >= _HALF)
                dstl_v[pl.ds(j16 * 16, 16)] = jnp.where(oob, _HALF + iota, d16)
            pltpu.sync_copy(ew_v.at[pl.ds(so, _SUB)], acc.at[dstl_v],
                            add=True)
            return ()

        lax.fori_loop(0, _BLK // _SUB, sub_body, (), unroll=False)
        return ()

    lax.fori_loop(0, _EPT // _BLK, blk_body, (), unroll=False)
    plsc.subcore_barrier()

    rb = sid * _TROWS_D
    last = _NS - 1
    tail = _HALF - last * _TROWS_D

    @pl.when(sid < last)
    def _():
        pltpu.sync_copy(acc.at[pl.ds(rb, _TROWS_D)], zb_v)
        pltpu.sync_copy(zb_v, out_h.at[pl.ds(base + rb, _TROWS_D)])

    @pl.when(sid == last)
    def _():
        pltpu.sync_copy(acc.at[pl.ds(rb, tail)], zb_v.at[pl.ds(0, tail)])
        pltpu.sync_copy(zb_v.at[pl.ds(0, tail)],
                        out_h.at[pl.ds(base + rb, tail)])


def _sc_deg(dstp, ewp):
    zrows = jnp.zeros((_TROWS_D,), jnp.float32)
    kfn = pl.kernel(
        _deg_body,
        out_type=jax.ShapeDtypeStruct((_N,), jnp.float32),
        mesh=_sc_mesh(),
        scratch_types=[
            pltpu.VMEM((_BLK,), jnp.int32),
            pltpu.VMEM((_BLK,), jnp.float32),
            pltpu.VMEM((_SUB,), jnp.int32),
            pltpu.VMEM((_TROWS_D,), jnp.float32),
            pltpu.VMEM_SHARED((_ACCR_D,), jnp.float32),
            pltpu.SemaphoreType.DMA,
        ],
        compiler_params=pltpu.CompilerParams(use_tc_tiling_on_sc=False),
    )
    return kfn(dstp, ewp, zrows)


def _row_spec(c):
    return pl.BlockSpec((_R, c), lambda i: (i, 0))


def _full_spec(shape):
    nd = len(shape)
    return pl.BlockSpec(shape, lambda i: (0,) * nd)


def _stage0_body(x_ref, deg_ref, encW_ref, encb_ref, gcnW_ref, resW_ref,
                 resb_ref, g_ref, id_ref):
    dinv = jax.lax.rsqrt(deg_ref[...] + 1.0)
    h = jnp.maximum(
        jax.lax.dot_general(x_ref[...], encW_ref[...], (((1,), (0,)), ((), ())),
                            preferred_element_type=jnp.float32) + encb_ref[...],
        0.0)
    g_ref[...] = dinv * jax.lax.dot_general(
        h, gcnW_ref[...], (((1,), (0,)), ((), ())),
        preferred_element_type=jnp.float32)
    id_ref[...] = jax.lax.dot_general(
        h, resW_ref[...], (((1,), (0,)), ((), ())),
        preferred_element_type=jnp.float32) + resb_ref[...]


def _stage_mid_body(scat_ref, g_ref, idc_ref, deg_ref, bprev_ref, gcnW_ref,
                    resW_ref, resb_ref, gout_ref, idout_ref):
    dinv = jax.lax.rsqrt(deg_ref[...] + 1.0)
    h = jnp.maximum(
        dinv * (scat_ref[...] + g_ref[...]) + bprev_ref[...] + idc_ref[...],
        0.0)
    gout_ref[...] = dinv * jax.lax.dot_general(
        h, gcnW_ref[...], (((1,), (0,)), ((), ())),
        preferred_element_type=jnp.float32)
    idout_ref[...] = jax.lax.dot_general(
        h, resW_ref[...], (((1,), (0,)), ((), ())),
        preferred_element_type=jnp.float32) + resb_ref[...]


def _stage_final_body(scat_ref, g_ref, idc_ref, deg_ref, bprev_ref, d1W_ref,
                      d1b_ref, d2W_ref, d2b_ref, out_ref):
    dinv = jax.lax.rsqrt(deg_ref[...] + 1.0)
    h = jnp.maximum(
        dinv * (scat_ref[...] + g_ref[...]) + bprev_ref[...] + idc_ref[...],
        0.0)
    p = jnp.maximum(
        jax.lax.dot_general(h, d1W_ref[...], (((1,), (0,)), ((), ())),
                            preferred_element_type=jnp.float32) + d1b_ref[...],
        0.0)
    out_ref[...] = jax.lax.dot_general(
        p, d2W_ref[...], (((1,), (0,)), ((), ())),
        preferred_element_type=jnp.float32) + d2b_ref[...]


def _stage0(x2, deg2, enc_W, enc_b, gcn_W0, res_W0, res_b0):
    n2 = x2.shape[0]
    c_in, c_h = enc_W.shape
    c_out = gcn_W0.shape[1]
    return pl.pallas_call(
        _stage0_body,
        grid=(n2 // _R,),
        in_specs=[
            _row_spec(c_in), _row_spec(1),
            _full_spec(enc_W.shape), _full_spec((1, c_h)),
            _full_spec(gcn_W0.shape), _full_spec(res_W0.shape),
            _full_spec((1, c_out)),
        ],
        out_specs=[_row_spec(c_out), _row_spec(c_out)],
        out_shape=[
            jax.ShapeDtypeStruct((n2, c_out), jnp.float32),
            jax.ShapeDtypeStruct((n2, c_out), jnp.float32),
        ],
    )(x2, deg2, enc_W, enc_b.reshape(1, -1), gcn_W0, res_W0,
      res_b0.reshape(1, -1))


def _stage_mid(scat, g, idc, deg2, b_prev, gcn_Wi, res_Wi, res_bi):
    n2, c_in = g.shape
    c_out = gcn_Wi.shape[1]
    return pl.pallas_call(
        _stage_mid_body,
        grid=(n2 // _R,),
        in_specs=[
            _row_spec(c_in), _row_spec(c_in), _row_spec(c_in), _row_spec(1),
            _full_spec((1, c_in)),
            _full_spec(gcn_Wi.shape), _full_spec(res_Wi.shape),
            _full_spec((1, c_out)),
        ],
        out_specs=[_row_spec(c_out), _row_spec(c_out)],
        out_shape=[
            jax.ShapeDtypeStruct((n2, c_out), jnp.float32),
            jax.ShapeDtypeStruct((n2, c_out), jnp.float32),
        ],
    )(scat, g, idc, deg2, b_prev.reshape(1, -1), gcn_Wi, res_Wi,
      res_bi.reshape(1, -1))


def _stage_final(scat, g, idc, deg2, b_prev, d1_W, d1_b, d2_W, d2_b):
    n2, c_in = g.shape
    return pl.pallas_call(
        _stage_final_body,
        grid=(n2 // _R,),
        in_specs=[
            _row_spec(c_in), _row_spec(c_in), _row_spec(c_in), _row_spec(1),
            _full_spec((1, c_in)),
            _full_spec(d1_W.shape), _full_spec((1, d1_W.shape[1])),
            _full_spec(d2_W.shape), _full_spec((1, 1)),
        ],
        out_specs=[_row_spec(1)],
        out_shape=[jax.ShapeDtypeStruct((n2, 1), jnp.float32)],
    )(scat, g, idc, deg2, b_prev.reshape(1, -1), d1_W, d1_b.reshape(1, -1),
      d2_W, d2_b.reshape(1, -1))[0]


def kernel(x, edge_index, edge_weight, enc_W, enc_b, gcn_W, gcn_b, res_W,
           res_b, d1_W, d1_b, d2_W, d2_b):
    gcn_W = [gcn_W[0], jnp.pad(gcn_W[1], ((0, 0), (0, 8))),
             jnp.pad(gcn_W[2], ((0, 8), (0, 0)))]
    gcn_b = [gcn_b[0], jnp.pad(gcn_b[1], (0, 8)), gcn_b[2]]
    res_W = [res_W[0], jnp.pad(res_W[1], ((0, 0), (0, 8))),
             jnp.pad(res_W[2], ((0, 8), (0, 0)))]
    res_b = [res_b[0], jnp.pad(res_b[1], (0, 8)), res_b[2]]

    src, dst = edge_index[0], edge_index[1]
    pad_idx = (jnp.arange(_EPAD - _E, dtype=jnp.int32) * 97) % _N
    srcp = jnp.concatenate([src, pad_idx])
    dstp = jnp.concatenate([dst, pad_idx])
    ewp = jnp.concatenate([edge_weight,
                           jnp.zeros((_EPAD - _E,), jnp.float32)])

    deg = _sc_deg(dstp, ewp)
    deg2 = jnp.tile(deg, 2).reshape(2 * _N, 1)

    x2 = x.reshape(2 * _N, x.shape[-1])
    g, idc = _stage0(x2, deg2, enc_W, enc_b, gcn_W[0], res_W[0], res_b[0])

    for i in range(3):
        c = g.shape[1]
        scat = jnp.concatenate([
            _sc_scatter_rows(g[:_N], srcp, dstp, ewp, c),
            _sc_scatter_rows(g[_N:], srcp, dstp, ewp, c),
        ], axis=0)
        if i < 2:
            g, idc = _stage_mid(scat, g, idc, deg2, gcn_b[i], gcn_W[i + 1],
                                res_W[i + 1], res_b[i + 1])
        else:
            pred = _stage_final(scat, g, idc, deg2, gcn_b[i], d1_W, d1_b,
                                d2_W, d2_b)
    return pred.reshape(_B, _N)


# speedup vs baseline: 24.0052x; 1.4749x over previous
import functools
import jax
import jax.numpy as jnp
from jax import lax
from jax.experimental import pallas as pl
from jax.experimental.pallas import tpu as pltpu
from jax.experimental.pallas import tpu_sc as plsc

_N = 100000
_B = 2
_R = 2000

_E = 1600000
_NC, _NS = 2, 16
_HALF = _N // _NC
_TROWS = 3128
_ACCR = _NS * _TROWS
_ZR = 184
_TROWS_D = 3128
_ACCR_D = _NS * _TROWS_D
_BLK = 2048
_SUB = 128
_EPT = 102400
_EPAD = _NS * _EPT


def _sc_mesh():
    return plsc.VectorSubcoreMesh(core_axis_name="c", subcore_axis_name="s",
                                  num_cores=_NC, num_subcores=_NS)


def _scatter_rows_body(c, g_h, src_h, dst_h, ew_h, z_h, out_h, src_v, dst_v,
                       ew_v, rows_a, rows_b, dstl_v, zb_v, acc, sem_a, sem_b):
    cid = lax.axis_index("c")
    sid = lax.axis_index("s")
    base = cid * _HALF
    tbase = sid * _EPT
    iota = lax.iota(jnp.int32, 16)

    pltpu.sync_copy(z_h, zb_v)
    for k in range(_TROWS // _ZR):
        pltpu.sync_copy(zb_v, acc.at[pl.ds(sid * _TROWS + k * _ZR, _ZR)])
    plsc.subcore_barrier()

    def process(rows_v, so):
        def mul_body(r, _):
            w = ew_v[pl.ds(so + r, 16)][0]
            for c16 in range(c // 16):
                sl = pl.ds(c16 * 16, 16)
                rows_v[r, sl] = rows_v[r, sl] * w
            return ()

        lax.fori_loop(0, _SUB, mul_body, (), unroll=16)
        for j16 in range(_SUB // 16):
            d16 = dst_v[pl.ds(so + j16 * 16, 16)] - base
            oob = (d16 < 0) | (d16 >= _HALF)
            dstl_v[pl.ds(j16 * 16, 16)] = jnp.where(oob, _HALF + iota, d16)
        pltpu.sync_copy(rows_v, acc.at[dstl_v], add=True)

    npair = _BLK // _SUB // 2

    def blk_body(j, _):
        off = tbase + j * _BLK
        pltpu.sync_copy(src_h.at[pl.ds(off, _BLK)], src_v)
        pltpu.sync_copy(dst_h.at[pl.ds(off, _BLK)], dst_v)
        pltpu.sync_copy(ew_h.at[pl.ds(off, _BLK)], ew_v.at[pl.ds(0, _BLK)])

        pltpu.async_copy(g_h.at[src_v.at[pl.ds(0, _SUB)]], rows_a, sem_a)

        def pair_body(t, _):
            s0 = 2 * t
            pltpu.async_copy(g_h.at[src_v.at[pl.ds((s0 + 1) * _SUB, _SUB)]],
                             rows_b, sem_b)
            pltpu.make_async_copy(g_h.at[pl.ds(0, _SUB)], rows_a,
                                  sem_a).wait()
            process(rows_a, s0 * _SUB)

            @pl.when(t < npair - 1)
            def _():
                pltpu.async_copy(
                    g_h.at[src_v.at[pl.ds((s0 + 2) * _SUB, _SUB)]], rows_a,
                    sem_a)

            pltpu.make_async_copy(g_h.at[pl.ds(0, _SUB)], rows_b,
                                  sem_b).wait()
            process(rows_b, (s0 + 1) * _SUB)
            return ()

        lax.fori_loop(0, npair, pair_body, (), unroll=False)
        return ()

    lax.fori_loop(0, _EPT // _BLK, blk_body, (), unroll=False)
    plsc.subcore_barrier()

    rb = sid * _TROWS
    last = _NS - 1
    nchunk = _TROWS // _ZR
    last_full = (_HALF - last * _TROWS) // _ZR
    tail_rows = _HALF - last * _TROWS - last_full * _ZR
    for k in range(nchunk):
        off = rb + k * _ZR
        if k < last_full:
            pltpu.sync_copy(acc.at[pl.ds(off, _ZR)], zb_v)
            pltpu.sync_copy(zb_v, out_h.at[pl.ds(base + off, _ZR)])
        elif k == last_full:

            @pl.when(sid < last)
            def _():
                pltpu.sync_copy(acc.at[pl.ds(off, _ZR)], zb_v)
                pltpu.sync_copy(zb_v, out_h.at[pl.ds(base + off, _ZR)])

            @pl.when(sid == last)
            def _():
                pltpu.sync_copy(acc.at[pl.ds(off, tail_rows)],
                                zb_v.at[pl.ds(0, tail_rows)])
                pltpu.sync_copy(zb_v.at[pl.ds(0, tail_rows)],
                                out_h.at[pl.ds(base + off, tail_rows)])
        else:

            @pl.when(sid < last)
            def _():
                pltpu.sync_copy(acc.at[pl.ds(off, _ZR)], zb_v)
                pltpu.sync_copy(zb_v, out_h.at[pl.ds(base + off, _ZR)])


def _sc_scatter_rows(g, srcp, dstp, ewp, c):
    zrows = jnp.zeros((_ZR, c), jnp.float32)
    kfn = functools.partial(
        pl.kernel,
        out_type=jax.ShapeDtypeStruct((_N, c), jnp.float32),
        mesh=_sc_mesh(),
        scratch_types=[
            pltpu.VMEM((_BLK,), jnp.int32),
            pltpu.VMEM((_BLK,), jnp.int32),
            pltpu.VMEM((_BLK + 16,), jnp.float32),
            pltpu.VMEM((_SUB, c), jnp.float32),
            pltpu.VMEM((_SUB, c), jnp.float32),
            pltpu.VMEM((_SUB,), jnp.int32),
            pltpu.VMEM((_ZR, c), jnp.float32),
            pltpu.VMEM_SHARED((_ACCR, c), jnp.float32),
            pltpu.SemaphoreType.DMA,
            pltpu.SemaphoreType.DMA,
        ],
        compiler_params=pltpu.CompilerParams(use_tc_tiling_on_sc=False),
    )(functools.partial(_scatter_rows_body, c))
    return kfn(g, srcp, dstp, ewp, zrows)


def _deg_body(dst_h, ew_h, z_h, out_h, dst_v, ew_v, dstl_v, zb_v, acc, sem):
    del sem
    cid = lax.axis_index("c")
    sid = lax.axis_index("s")
    base = cid * _HALF
    tbase = sid * _EPT
    iota = lax.iota(jnp.int32, 16)

    pltpu.sync_copy(z_h, zb_v)
    pltpu.sync_copy(zb_v, acc.at[pl.ds(sid * _TROWS_D, _TROWS_D)])
    plsc.subcore_barrier()

    def blk_body(j, _):
        off = tbase + j * _BLK
        pltpu.sync_copy(dst_h.at[pl.ds(off, _BLK)], dst_v)
        pltpu.sync_copy(ew_h.at[pl.ds(off, _BLK)], ew_v)

        def sub_body(s, _):
            so = s * _SUB
            for j16 in range(_SUB // 16):
                d16 = dst_v[pl.ds(so + j16 * 16, 16)] - base
                oob = (d16 < 0) | (d16 >= _HALF)
                dstl_v[pl.ds(j16 * 16, 16)] = jnp.where(oob, _HALF + iota, d16)
            pltpu.sync_copy(ew_v.at[pl.ds(so, _SUB)], acc.at[dstl_v],
                            add=True)
            return ()

        lax.fori_loop(0, _BLK // _SUB, sub_body, (), unroll=False)
        return ()

    lax.fori_loop(0, _EPT // _BLK, blk_body, (), unroll=False)
    plsc.subcore_barrier()

    rb = sid * _TROWS_D
    last = _NS - 1
    tail = _HALF - last * _TROWS_D

    @pl.when(sid < last)
    def _():
        pltpu.sync_copy(acc.at[pl.ds(rb, _TROWS_D)], zb_v)
        pltpu.sync_copy(zb_v, out_h.at[pl.ds(base + rb, _TROWS_D)])

    @pl.when(sid == last)
    def _():
        pltpu.sync_copy(acc.at[pl.ds(rb, tail)], zb_v.at[pl.ds(0, tail)])
        pltpu.sync_copy(zb_v.at[pl.ds(0, tail)],
                        out_h.at[pl.ds(base + rb, tail)])


def _sc_deg(dstp, ewp):
    zrows = jnp.zeros((_TROWS_D,), jnp.float32)
    kfn = pl.kernel(
        _deg_body,
        out_type=jax.ShapeDtypeStruct((_N,), jnp.float32),
        mesh=_sc_mesh(),
        scratch_types=[
            pltpu.VMEM((_BLK,), jnp.int32),
            pltpu.VMEM((_BLK,), jnp.float32),
            pltpu.VMEM((_SUB,), jnp.int32),
            pltpu.VMEM((_TROWS_D,), jnp.float32),
            pltpu.VMEM_SHARED((_ACCR_D,), jnp.float32),
            pltpu.SemaphoreType.DMA,
        ],
        compiler_params=pltpu.CompilerParams(use_tc_tiling_on_sc=False),
    )
    return kfn(dstp, ewp, zrows)


def _row_spec(c):
    return pl.BlockSpec((_R, c), lambda i: (i, 0))


def _full_spec(shape):
    nd = len(shape)
    return pl.BlockSpec(shape, lambda i: (0,) * nd)


def _stage0_body(x_ref, deg_ref, encW_ref, encb_ref, gcnW_ref, resW_ref,
                 resb_ref, g_ref, id_ref):
    dinv = jax.lax.rsqrt(deg_ref[...] + 1.0)
    h = jnp.maximum(
        jax.lax.dot_general(x_ref[...], encW_ref[...], (((1,), (0,)), ((), ())),
                            preferred_element_type=jnp.float32) + encb_ref[...],
        0.0)
    g_ref[...] = dinv * jax.lax.dot_general(
        h, gcnW_ref[...], (((1,), (0,)), ((), ())),
        preferred_element_type=jnp.float32)
    id_ref[...] = jax.lax.dot_general(
        h, resW_ref[...], (((1,), (0,)), ((), ())),
        preferred_element_type=jnp.float32) + resb_ref[...]


def _stage_mid_body(scat_ref, g_ref, idc_ref, deg_ref, bprev_ref, gcnW_ref,
                    resW_ref, resb_ref, gout_ref, idout_ref):
    dinv = jax.lax.rsqrt(deg_ref[...] + 1.0)
    h = jnp.maximum(
        dinv * (scat_ref[...] + g_ref[...]) + bprev_ref[...] + idc_ref[...],
        0.0)
    gout_ref[...] = dinv * jax.lax.dot_general(
        h, gcnW_ref[...], (((1,), (0,)), ((), ())),
        preferred_element_type=jnp.float32)
    idout_ref[...] = jax.lax.dot_general(
        h, resW_ref[...], (((1,), (0,)), ((), ())),
        preferred_element_type=jnp.float32) + resb_ref[...]


def _stage_final_body(scat_ref, g_ref, idc_ref, deg_ref, bprev_ref, d1W_ref,
                      d1b_ref, d2W_ref, d2b_ref, out_ref):
    dinv = jax.lax.rsqrt(deg_ref[...] + 1.0)
    h = jnp.maximum(
        dinv * (scat_ref[...] + g_ref[...]) + bprev_ref[...] + idc_ref[...],
        0.0)
    p = jnp.maximum(
        jax.lax.dot_general(h, d1W_ref[...], (((1,), (0,)), ((), ())),
                            preferred_element_type=jnp.float32) + d1b_ref[...],
        0.0)
    out_ref[...] = jax.lax.dot_general(
        p, d2W_ref[...], (((1,), (0,)), ((), ())),
        preferred_element_type=jnp.float32) + d2b_ref[...]


def _stage0(x2, deg2, enc_W, enc_b, gcn_W0, res_W0, res_b0):
    n2 = x2.shape[0]
    c_in, c_h = enc_W.shape
    c_out = gcn_W0.shape[1]
    return pl.pallas_call(
        _stage0_body,
        grid=(n2 // _R,),
        in_specs=[
            _row_spec(c_in), _row_spec(1),
            _full_spec(enc_W.shape), _full_spec((1, c_h)),
            _full_spec(gcn_W0.shape), _full_spec(res_W0.shape),
            _full_spec((1, c_out)),
        ],
        out_specs=[_row_spec(c_out), _row_spec(c_out)],
        out_shape=[
            jax.ShapeDtypeStruct((n2, c_out), jnp.float32),
            jax.ShapeDtypeStruct((n2, c_out), jnp.float32),
        ],
    )(x2, deg2, enc_W, enc_b.reshape(1, -1), gcn_W0, res_W0,
      res_b0.reshape(1, -1))


def _stage_mid(scat, g, idc, deg2, b_prev, gcn_Wi, res_Wi, res_bi):
    n2, c_in = g.shape
    c_out = gcn_Wi.shape[1]
    return pl.pallas_call(
        _stage_mid_body,
        grid=(n2 // _R,),
        in_specs=[
            _row_spec(c_in), _row_spec(c_in), _row_spec(c_in), _row_spec(1),
            _full_spec((1, c_in)),
            _full_spec(gcn_Wi.shape), _full_spec(res_Wi.shape),
            _full_spec((1, c_out)),
        ],
        out_specs=[_row_spec(c_out), _row_spec(c_out)],
        out_shape=[
            jax.ShapeDtypeStruct((n2, c_out), jnp.float32),
            jax.ShapeDtypeStruct((n2, c_out), jnp.float32),
        ],
    )(scat, g, idc, deg2, b_prev.reshape(1, -1), gcn_Wi, res_Wi,
      res_bi.reshape(1, -1))


def _stage_final(scat, g, idc, deg2, b_prev, d1_W, d1_b, d2_W, d2_b):
    n2, c_in = g.shape
    return pl.pallas_call(
        _stage_final_body,
        grid=(n2 // _R,),
        in_specs=[
            _row_spec(c_in), _row_spec(c_in), _row_spec(c_in), _row_spec(1),
            _full_spec((1, c_in)),
            _full_spec(d1_W.shape), _full_spec((1, d1_W.shape[1])),
            _full_spec(d2_W.shape), _full_spec((1, 1)),
        ],
        out_specs=[_row_spec(1)],
        out_shape=[jax.ShapeDtypeStruct((n2, 1), jnp.float32)],
    )(scat, g, idc, deg2, b_prev.reshape(1, -1), d1_W, d1_b.reshape(1, -1),
      d2_W, d2_b.reshape(1, -1))[0]


def kernel(x, edge_index, edge_weight, enc_W, enc_b, gcn_W, gcn_b, res_W,
           res_b, d1_W, d1_b, d2_W, d2_b):
    gcn_W = [gcn_W[0], jnp.pad(gcn_W[1], ((0, 0), (0, 8))),
             jnp.pad(gcn_W[2], ((0, 8), (0, 0)))]
    gcn_b = [gcn_b[0], jnp.pad(gcn_b[1], (0, 8)), gcn_b[2]]
    res_W = [res_W[0], jnp.pad(res_W[1], ((0, 0), (0, 8))),
             jnp.pad(res_W[2], ((0, 8), (0, 0)))]
    res_b = [res_b[0], jnp.pad(res_b[1], (0, 8)), res_b[2]]

    src, dst = edge_index[0], edge_index[1]
    pad_idx = (jnp.arange(_EPAD - _E, dtype=jnp.int32) * 97) % _N
    srcp = jnp.concatenate([src, pad_idx])
    dstp = jnp.concatenate([dst, pad_idx])
    ewp = jnp.concatenate([edge_weight,
                           jnp.zeros((_EPAD - _E,), jnp.float32)])

    deg = _sc_deg(dstp, ewp)
    deg2 = jnp.tile(deg, 2).reshape(2 * _N, 1)

    x2 = x.reshape(2 * _N, x.shape[-1])
    g, idc = _stage0(x2, deg2, enc_W, enc_b, gcn_W[0], res_W[0], res_b[0])

    for i in range(3):
        c = g.shape[1]
        scat = jnp.concatenate([
            _sc_scatter_rows(g[:_N], srcp, dstp, ewp, c),
            _sc_scatter_rows(g[_N:], srcp, dstp, ewp, c),
        ], axis=0)
        if i < 2:
            g, idc = _stage_mid(scat, g, idc, deg2, gcn_b[i], gcn_W[i + 1],
                                res_W[i + 1], res_b[i + 1])
        else:
            pred = _stage_final(scat, g, idc, deg2, gcn_b[i], d1_W, d1_b,
                                d2_W, d2_b)
    return pred.reshape(_B, _N)


# speedup vs baseline: 27.6524x; 1.1519x over previous
import functools
import jax
import jax.numpy as jnp
from jax import lax
from jax.experimental import pallas as pl
from jax.experimental.pallas import tpu as pltpu
from jax.experimental.pallas import tpu_sc as plsc

_N = 100000
_B = 2
_R = 2000

_E = 1600000
_NC, _NS = 2, 16
_HALF = _N // _NC
_TROWS = 3128
_ACCR = _NS * _TROWS
_ZR = 184
_TROWS_D = 3128
_ACCR_D = _NS * _TROWS_D
_BLK = 3072
_SUB = 128
_EPT = 104448
_EPAD = _NS * _EPT


def _sc_mesh():
    return plsc.VectorSubcoreMesh(core_axis_name="c", subcore_axis_name="s",
                                  num_cores=_NC, num_subcores=_NS)


def _scatter_rows_body(c, g_h, src_h, dst_h, ew_h, z_h, out_h, src_v, dst_v,
                       ew_v, rows_a, rows_b, rows_c, dstl_a, dstl_b, dstl_c,
                       zb_v, acc, ga, gb, gc, sa, sb, sc):
    cid = lax.axis_index("c")
    sid = lax.axis_index("s")
    base = cid * _HALF
    tbase = sid * _EPT
    iota = lax.iota(jnp.int32, 16)

    pltpu.sync_copy(z_h, zb_v)
    for k in range(_TROWS // _ZR):
        pltpu.sync_copy(zb_v, acc.at[pl.ds(sid * _TROWS + k * _ZR, _ZR)])
    plsc.subcore_barrier()

    rows = [rows_a, rows_b, rows_c]
    dstl = [dstl_a, dstl_b, dstl_c]
    gsem = [ga, gb, gc]
    ssem = [sa, sb, sc]

    def sem_wait(sem, buf):
        pltpu.make_async_copy(g_h.at[pl.ds(0, _SUB)], buf, sem).wait()

    def compute(rv, dl, so):
        def mul_body(r, _):
            w = ew_v[pl.ds(so + r, 16)][0]
            for c16 in range(c // 16):
                sl = pl.ds(c16 * 16, 16)
                rv[r, sl] = rv[r, sl] * w
            return ()

        lax.fori_loop(0, _SUB, mul_body, (), unroll=16)
        for j16 in range(_SUB // 16):
            d16 = dst_v[pl.ds(so + j16 * 16, 16)] - base
            oob = (d16 < 0) | (d16 >= _HALF)
            dl[pl.ds(j16 * 16, 16)] = jnp.where(oob, _HALF + iota, d16)

    nch = _BLK // _SUB

    def blk_body(j, _):
        off = tbase + j * _BLK
        pltpu.sync_copy(src_h.at[pl.ds(off, _BLK)], src_v)
        pltpu.sync_copy(dst_h.at[pl.ds(off, _BLK)], dst_v)
        pltpu.sync_copy(ew_h.at[pl.ds(off, _BLK)], ew_v.at[pl.ds(0, _BLK)])

        pltpu.async_copy(g_h.at[src_v.at[pl.ds(0, _SUB)]], rows[0], gsem[0])
        pltpu.async_copy(g_h.at[src_v.at[pl.ds(_SUB, _SUB)]], rows[1],
                         gsem[1])

        def trip_body(t, _):
            for q in range(3):
                s_idx = 3 * t + q
                r = (q + 2) % 3
                sem_wait(gsem[q], rows[q])
                compute(rows[q], dstl[q], s_idx * _SUB)
                pltpu.async_copy(rows[q], acc.at[dstl[q]], ssem[q], add=True)
                if q == 0:

                    @pl.when(jnp.logical_or(j > 0, t > 0))
                    def _():
                        sem_wait(ssem[r], rows[r])
                else:
                    sem_wait(ssem[r], rows[r])

                @pl.when(s_idx + 2 < nch)
                def _():
                    pltpu.async_copy(
                        g_h.at[src_v.at[pl.ds((s_idx + 2) * _SUB, _SUB)]],
                        rows[r], gsem[r])

            return ()

        lax.fori_loop(0, nch // 3, trip_body, (), unroll=False)
        return ()

    lax.fori_loop(0, _EPT // _BLK, blk_body, (), unroll=False)
    sem_wait(ssem[2], rows[2])
    plsc.subcore_barrier()

    rb = sid * _TROWS
    last = _NS - 1
    nchunk = _TROWS // _ZR
    last_full = (_HALF - last * _TROWS) // _ZR
    tail_rows = _HALF - last * _TROWS - last_full * _ZR
    for k in range(nchunk):
        off = rb + k * _ZR
        if k < last_full:
            pltpu.sync_copy(acc.at[pl.ds(off, _ZR)], zb_v)
            pltpu.sync_copy(zb_v, out_h.at[pl.ds(base + off, _ZR)])
        elif k == last_full:

            @pl.when(sid < last)
            def _():
                pltpu.sync_copy(acc.at[pl.ds(off, _ZR)], zb_v)
                pltpu.sync_copy(zb_v, out_h.at[pl.ds(base + off, _ZR)])

            @pl.when(sid == last)
            def _():
                pltpu.sync_copy(acc.at[pl.ds(off, tail_rows)],
                                zb_v.at[pl.ds(0, tail_rows)])
                pltpu.sync_copy(zb_v.at[pl.ds(0, tail_rows)],
                                out_h.at[pl.ds(base + off, tail_rows)])
        else:

            @pl.when(sid < last)
            def _():
                pltpu.sync_copy(acc.at[pl.ds(off, _ZR)], zb_v)
                pltpu.sync_copy(zb_v, out_h.at[pl.ds(base + off, _ZR)])


def _sc_scatter_rows(g, srcp, dstp, ewp, c):
    zrows = jnp.zeros((_ZR, c), jnp.float32)
    kfn = functools.partial(
        pl.kernel,
        out_type=jax.ShapeDtypeStruct((_N, c), jnp.float32),
        mesh=_sc_mesh(),
        scratch_types=[
            pltpu.VMEM((_BLK,), jnp.int32),
            pltpu.VMEM((_BLK,), jnp.int32),
            pltpu.VMEM((_BLK + 16,), jnp.float32),
            pltpu.VMEM((_SUB, c), jnp.float32),
            pltpu.VMEM((_SUB, c), jnp.float32),
            pltpu.VMEM((_SUB, c), jnp.float32),
            pltpu.VMEM((_SUB,), jnp.int32),
            pltpu.VMEM((_SUB,), jnp.int32),
            pltpu.VMEM((_SUB,), jnp.int32),
            pltpu.VMEM((_ZR, c), jnp.float32),
            pltpu.VMEM_SHARED((_ACCR, c), jnp.float32),
            pltpu.SemaphoreType.DMA,
            pltpu.SemaphoreType.DMA,
            pltpu.SemaphoreType.DMA,
            pltpu.SemaphoreType.DMA,
            pltpu.SemaphoreType.DMA,
            pltpu.SemaphoreType.DMA,
        ],
        compiler_params=pltpu.CompilerParams(use_tc_tiling_on_sc=False),
    )(functools.partial(_scatter_rows_body, c))
    return kfn(g, srcp, dstp, ewp, zrows)


def _deg_body(dst_h, ew_h, z_h, out_h, dst_v, ew_v, dstl_v, zb_v, acc, sem):
    del sem
    cid = lax.axis_index("c")
    sid = lax.axis_index("s")
    base = cid * _HALF
    tbase = sid * _EPT
    iota = lax.iota(jnp.int32, 16)

    pltpu.sync_copy(z_h, zb_v)
    pltpu.sync_copy(zb_v, acc.at[pl.ds(sid * _TROWS_D, _TROWS_D)])
    plsc.subcore_barrier()

    def blk_body(j, _):
        off = tbase + j * _BLK
        pltpu.sync_copy(dst_h.at[pl.ds(off, _BLK)], dst_v)
        pltpu.sync_copy(ew_h.at[pl.ds(off, _BLK)], ew_v)

        def sub_body(s, _):
            so = s * _SUB
            for j16 in range(_SUB // 16):
                d16 = dst_v[pl.ds(so + j16 * 16, 16)] - base
                oob = (d16 < 0) | (d16 >= _HALF)
                dstl_v[pl.ds(j16 * 16, 16)] = jnp.where(oob, _HALF + iota, d16)
            pltpu.sync_copy(ew_v.at[pl.ds(so, _SUB)], acc.at[dstl_v],
                            add=True)
            return ()

        lax.fori_loop(0, _BLK // _SUB, sub_body, (), unroll=False)
        return ()

    lax.fori_loop(0, _EPT // _BLK, blk_body, (), unroll=False)
    plsc.subcore_barrier()

    rb = sid * _TROWS_D
    last = _NS - 1
    tail = _HALF - last * _TROWS_D

    @pl.when(sid < last)
    def _():
        pltpu.sync_copy(acc.at[pl.ds(rb, _TROWS_D)], zb_v)
        pltpu.sync_copy(zb_v, out_h.at[pl.ds(base + rb, _TROWS_D)])

    @pl.when(sid == last)
    def _():
        pltpu.sync_copy(acc.at[pl.ds(rb, tail)], zb_v.at[pl.ds(0, tail)])
        pltpu.sync_copy(zb_v.at[pl.ds(0, tail)],
                        out_h.at[pl.ds(base + rb, tail)])


def _sc_deg(dstp, ewp):
    zrows = jnp.zeros((_TROWS_D,), jnp.float32)
    kfn = pl.kernel(
        _deg_body,
        out_type=jax.ShapeDtypeStruct((_N,), jnp.float32),
        mesh=_sc_mesh(),
        scratch_types=[
            pltpu.VMEM((_BLK,), jnp.int32),
            pltpu.VMEM((_BLK,), jnp.float32),
            pltpu.VMEM((_SUB,), jnp.int32),
            pltpu.VMEM((_TROWS_D,), jnp.float32),
            pltpu.VMEM_SHARED((_ACCR_D,), jnp.float32),
            pltpu.SemaphoreType.DMA,
        ],
        compiler_params=pltpu.CompilerParams(use_tc_tiling_on_sc=False),
    )
    return kfn(dstp, ewp, zrows)


def _row_spec(c):
    return pl.BlockSpec((_R, c), lambda i: (i, 0))


def _full_spec(shape):
    nd = len(shape)
    return pl.BlockSpec(shape, lambda i: (0,) * nd)


def _stage0_body(x_ref, deg_ref, encW_ref, encb_ref, gcnW_ref, resW_ref,
                 resb_ref, g_ref, id_ref):
    dinv = jax.lax.rsqrt(deg_ref[...] + 1.0)
    h = jnp.maximum(
        jax.lax.dot_general(x_ref[...], encW_ref[...], (((1,), (0,)), ((), ())),
                            preferred_element_type=jnp.float32) + encb_ref[...],
        0.0)
    g_ref[...] = dinv * jax.lax.dot_general(
        h, gcnW_ref[...], (((1,), (0,)), ((), ())),
        preferred_element_type=jnp.float32)
    id_ref[...] = jax.lax.dot_general(
        h, resW_ref[...], (((1,), (0,)), ((), ())),
        preferred_element_type=jnp.float32) + resb_ref[...]


def _stage_mid_body(scat_ref, g_ref, idc_ref, deg_ref, bprev_ref, gcnW_ref,
                    resW_ref, resb_ref, gout_ref, idout_ref):
    dinv = jax.lax.rsqrt(deg_ref[...] + 1.0)
    h = jnp.maximum(
        dinv * (scat_ref[...] + g_ref[...]) + bprev_ref[...] + idc_ref[...],
        0.0)
    gout_ref[...] = dinv * jax.lax.dot_general(
        h, gcnW_ref[...], (((1,), (0,)), ((), ())),
        preferred_element_type=jnp.float32)
    idout_ref[...] = jax.lax.dot_general(
        h, resW_ref[...], (((1,), (0,)), ((), ())),
        preferred_element_type=jnp.float32) + resb_ref[...]


def _stage_final_body(scat_ref, g_ref, idc_ref, deg_ref, bprev_ref, d1W_ref,
                      d1b_ref, d2W_ref, d2b_ref, out_ref):
    dinv = jax.lax.rsqrt(deg_ref[...] + 1.0)
    h = jnp.maximum(
        dinv * (scat_ref[...] + g_ref[...]) + bprev_ref[...] + idc_ref[...],
        0.0)
    p = jnp.maximum(
        jax.lax.dot_general(h, d1W_ref[...], (((1,), (0,)), ((), ())),
                            preferred_element_type=jnp.float32) + d1b_ref[...],
        0.0)
    out_ref[...] = jax.lax.dot_general(
        p, d2W_ref[...], (((1,), (0,)), ((), ())),
        preferred_element_type=jnp.float32) + d2b_ref[...]


def _stage0(x2, deg2, enc_W, enc_b, gcn_W0, res_W0, res_b0):
    n2 = x2.shape[0]
    c_in, c_h = enc_W.shape
    c_out = gcn_W0.shape[1]
    return pl.pallas_call(
        _stage0_body,
        grid=(n2 // _R,),
        in_specs=[
            _row_spec(c_in), _row_spec(1),
            _full_spec(enc_W.shape), _full_spec((1, c_h)),
            _full_spec(gcn_W0.shape), _full_spec(res_W0.shape),
            _full_spec((1, c_out)),
        ],
        out_specs=[_row_spec(c_out), _row_spec(c_out)],
        out_shape=[
            jax.ShapeDtypeStruct((n2, c_out), jnp.float32),
            jax.ShapeDtypeStruct((n2, c_out), jnp.float32),
        ],
    )(x2, deg2, enc_W, enc_b.reshape(1, -1), gcn_W0, res_W0,
      res_b0.reshape(1, -1))


def _stage_mid(scat, g, idc, deg2, b_prev, gcn_Wi, res_Wi, res_bi):
    n2, c_in = g.shape
    c_out = gcn_Wi.shape[1]
    return pl.pallas_call(
        _stage_mid_body,
        grid=(n2 // _R,),
        in_specs=[
            _row_spec(c_in), _row_spec(c_in), _row_spec(c_in), _row_spec(1),
            _full_spec((1, c_in)),
            _full_spec(gcn_Wi.shape), _full_spec(res_Wi.shape),
            _full_spec((1, c_out)),
        ],
        out_specs=[_row_spec(c_out), _row_spec(c_out)],
        out_shape=[
            jax.ShapeDtypeStruct((n2, c_out), jnp.float32),
            jax.ShapeDtypeStruct((n2, c_out), jnp.float32),
        ],
    )(scat, g, idc, deg2, b_prev.reshape(1, -1), gcn_Wi, res_Wi,
      res_bi.reshape(1, -1))


def _stage_final(scat, g, idc, deg2, b_prev, d1_W, d1_b, d2_W, d2_b):
    n2, c_in = g.shape
    return pl.pallas_call(
        _stage_final_body,
        grid=(n2 // _R,),
        in_specs=[
            _row_spec(c_in), _row_spec(c_in), _row_spec(c_in), _row_spec(1),
            _full_spec((1, c_in)),
            _full_spec(d1_W.shape), _full_spec((1, d1_W.shape[1])),
            _full_spec(d2_W.shape), _full_spec((1, 1)),
        ],
        out_specs=[_row_spec(1)],
        out_shape=[jax.ShapeDtypeStruct((n2, 1), jnp.float32)],
    )(scat, g, idc, deg2, b_prev.reshape(1, -1), d1_W, d1_b.reshape(1, -1),
      d2_W, d2_b.reshape(1, -1))[0]


def kernel(x, edge_index, edge_weight, enc_W, enc_b, gcn_W, gcn_b, res_W,
           res_b, d1_W, d1_b, d2_W, d2_b):
    gcn_W = [gcn_W[0], jnp.pad(gcn_W[1], ((0, 0), (0, 8))),
             jnp.pad(gcn_W[2], ((0, 8), (0, 0)))]
    gcn_b = [gcn_b[0], jnp.pad(gcn_b[1], (0, 8)), gcn_b[2]]
    res_W = [res_W[0], jnp.pad(res_W[1], ((0, 0), (0, 8))),
             jnp.pad(res_W[2], ((0, 8), (0, 0)))]
    res_b = [res_b[0], jnp.pad(res_b[1], (0, 8)), res_b[2]]

    src, dst = edge_index[0], edge_index[1]
    pad_idx = (jnp.arange(_EPAD - _E, dtype=jnp.int32) * 97) % _N
    srcp = jnp.concatenate([src, pad_idx])
    dstp = jnp.concatenate([dst, pad_idx])
    ewp = jnp.concatenate([edge_weight,
                           jnp.zeros((_EPAD - _E,), jnp.float32)])

    deg = _sc_deg(dstp, ewp)
    deg2 = jnp.tile(deg, 2).reshape(2 * _N, 1)

    x2 = x.reshape(2 * _N, x.shape[-1])
    g, idc = _stage0(x2, deg2, enc_W, enc_b, gcn_W[0], res_W[0], res_b[0])

    for i in range(3):
        c = g.shape[1]
        scat = jnp.concatenate([
            _sc_scatter_rows(g[:_N], srcp, dstp, ewp, c),
            _sc_scatter_rows(g[_N:], srcp, dstp, ewp, c),
        ], axis=0)
        if i < 2:
            g, idc = _stage_mid(scat, g, idc, deg2, gcn_b[i], gcn_W[i + 1],
                                res_W[i + 1], res_b[i + 1])
        else:
            pred = _stage_final(scat, g, idc, deg2, gcn_b[i], d1_W, d1_b,
                                d2_W, d2_b)
    return pred.reshape(_B, _N)
